# baseline (speedup 1.0000x reference)
import functools

import jax
import jax.numpy as jnp
from jax import lax
from jax.experimental import pallas as pl
from jax.experimental.pallas import tpu as pltpu
from jax.experimental.pallas import tpu_sc as plsc

N_FACTORS = 64
L = 16

_info = plsc.get_sparse_core_info()
NC, NS = _info.num_cores, _info.num_subcores
NW = NC * NS

_IDX_CHUNK = 128
_TC_BLK = 65536


def _project_body(t_ref, w_ref, b_ref, o_ref):
    o_ref[...] = jnp.sum(t_ref[...] * w_ref[...], axis=0) + b_ref[0, 0]


def _tc_project(table_t, wcol, bias11, off_blk=0, n_out=None):
    f, n = table_t.shape
    if n_out is None:
        n_out = n
    grid = (n_out + _TC_BLK - 1) // _TC_BLK
    return pl.pallas_call(
        _project_body,
        grid=(grid,),
        compiler_params=pltpu.CompilerParams(
            vmem_limit_bytes=60 * 1024 * 1024),
        in_specs=[
            pl.BlockSpec((f, _TC_BLK), lambda i: (0, i + off_blk)),
            pl.BlockSpec((f, 1), lambda i: (0, 0)),
            pl.BlockSpec((1, 1), lambda i: (0, 0)),
        ],
        out_specs=pl.BlockSpec((_TC_BLK,), lambda i: (i,)),
        out_shape=jax.ShapeDtypeStruct((n_out,), jnp.float32),
    )(table_t, wcol, bias11)


@functools.cache
def _sc_pick(B, nu, nm):
    b_per_w = B // NW
    n_chunks = b_per_w // _IDX_CHUNK
    mesh = plsc.VectorSubcoreMesh(core_axis_name="c", subcore_axis_name="s")

    @functools.partial(
        pl.kernel,
        mesh=mesh,
        compiler_params=pltpu.CompilerParams(
            needs_layout_passes=False, use_tc_tiling_on_sc=False),
        out_type=jax.ShapeDtypeStruct((B,), jnp.float32),
        scratch_types=[
            pltpu.VMEM((b_per_w,), jnp.int32),
            pltpu.VMEM((b_per_w,), jnp.int32),
            pltpu.VMEM((b_per_w,), jnp.int32),
            pltpu.VMEM((b_per_w,), jnp.int32),
            pltpu.VMEM((b_per_w, L), jnp.float32),
            pltpu.VMEM((b_per_w, L), jnp.float32),
            pltpu.VMEM((b_per_w,), jnp.float32),
            pltpu.SemaphoreType.DMA,
        ],
    )
    def k(uidx_hbm, midx_hbm, pu_hbm, pm_hbm, out_hbm,
          uid_v, mid_v, uhi_v, mhi_v, ubuf, mbuf, out_v, sem):
        wid = lax.axis_index("s") * NC + lax.axis_index("c")
        base = wid * b_per_w
        pltpu.sync_copy(uidx_hbm.at[pl.ds(base, b_per_w)], uid_v)
        pltpu.sync_copy(midx_hbm.at[pl.ds(base, b_per_w)], mid_v)
        for c in range(b_per_w // L):
            sl = pl.ds(c * L, L)
            uhi_v[sl] = lax.shift_right_logical(uid_v[sl], 4)
            mhi_v[sl] = lax.shift_right_logical(mid_v[sl], 4)
        copies = []
        for c in range(n_chunks):
            sl = pl.ds(c * _IDX_CHUNK, _IDX_CHUNK)
            copies.append(
                pltpu.async_copy(pu_hbm.at[uhi_v.at[sl]], ubuf.at[sl], sem))
            copies.append(
                pltpu.async_copy(pm_hbm.at[mhi_v.at[sl]], mbuf.at[sl], sem))
        for cp in copies:
            cp.wait()
        lanes = lax.iota(jnp.int32, L)
        for g in range(b_per_w // L):
            sl = pl.ds(g * L, L)
            rowv = lanes + (g * L)
            ulo = lax.bitwise_and(uid_v[sl], L - 1)
            mlo = lax.bitwise_and(mid_v[sl], L - 1)
            pu = plsc.load_gather(ubuf, [rowv, ulo])
            pm = plsc.load_gather(mbuf, [rowv, mlo])
            out_v[sl] = pu + pm
        pltpu.sync_copy(out_v, out_hbm.at[pl.ds(base, b_per_w)])

    return k


def kernel(user, movie, user_table, movie_table, W, b):
    B = user.shape[0]
    tu = user_table.T
    tm = movie_table.T
    wu = W[0, :N_FACTORS].reshape(N_FACTORS, 1)
    wm = W[0, N_FACTORS:].reshape(N_FACTORS, 1)
    zero11 = jnp.zeros((1, 1), jnp.float32)
    pu = _tc_project(tu, wu, zero11)
    pm = _tc_project(tm, wm, b.reshape(1, 1))
    p2u = pu.reshape(-1, L)
    p2m = pm.reshape(-1, L)
    out = _sc_pick(B, p2u.shape[0], p2m.shape[0])(
        user.astype(jnp.int32), movie.astype(jnp.int32), p2u, p2m)
    return out.reshape(B, 1)


# speedup vs baseline: 1.0341x; 1.0341x over previous
import functools

import jax
import jax.numpy as jnp
from jax import lax
from jax.experimental import pallas as pl
from jax.experimental.pallas import tpu as pltpu
from jax.experimental.pallas import tpu_sc as plsc

N_FACTORS = 64
L = 16

_info = plsc.get_sparse_core_info()
NC, NS = _info.num_cores, _info.num_subcores
NW = NC * NS

_IDX_CHUNK = 128
_TC_BLK = 32768


def _project_body(t_ref, w_ref, b_ref, o_ref):
    o_ref[...] = jnp.sum(t_ref[...] * w_ref[...], axis=0) + b_ref[0, 0]


def _project_both(tu, tm, wu, wm, bias11):
    f, nu = tu.shape
    _, nm = tm.shape
    gu = (nu + _TC_BLK - 1) // _TC_BLK
    gm = (nm + _TC_BLK - 1) // _TC_BLK

    def body(tu_ref, tm_ref, wu_ref, wm_ref, b_ref, pu_ref, pm_ref):
        i = pl.program_id(0)

        @pl.when(i < gu)
        def _():
            pu_ref[...] = jnp.sum(tu_ref[...] * wu_ref[...], axis=0)

        @pl.when(i >= gu)
        def _():
            pm_ref[...] = (jnp.sum(tm_ref[...] * wm_ref[...], axis=0)
                           + b_ref[0, 0])

    return pl.pallas_call(
        body,
        grid=(gu + gm,),
        compiler_params=pltpu.CompilerParams(
            vmem_limit_bytes=60 * 1024 * 1024),
        in_specs=[
            pl.BlockSpec((f, _TC_BLK),
                         lambda i: (0, jnp.minimum(i, gu - 1))),
            pl.BlockSpec((f, _TC_BLK),
                         lambda i: (0, jnp.clip(i - gu, 0, gm - 1))),
            pl.BlockSpec((f, 1), lambda i: (0, 0)),
            pl.BlockSpec((f, 1), lambda i: (0, 0)),
            pl.BlockSpec((1, 1), lambda i: (0, 0)),
        ],
        out_specs=[
            pl.BlockSpec((_TC_BLK,), lambda i: (jnp.minimum(i, gu - 1),)),
            pl.BlockSpec((_TC_BLK,), lambda i: (jnp.clip(i - gu, 0, gm - 1),)),
        ],
        out_shape=[
            jax.ShapeDtypeStruct((nu,), jnp.float32),
            jax.ShapeDtypeStruct((nm,), jnp.float32),
        ],
    )(tu, tm, wu, wm, bias11)


def _tc_project(table_t, wcol, bias11, off_blk=0, n_out=None):
    f, n = table_t.shape
    if n_out is None:
        n_out = n
    grid = (n_out + _TC_BLK - 1) // _TC_BLK
    return pl.pallas_call(
        _project_body,
        grid=(grid,),
        compiler_params=pltpu.CompilerParams(
            vmem_limit_bytes=60 * 1024 * 1024),
        in_specs=[
            pl.BlockSpec((f, _TC_BLK), lambda i: (0, i + off_blk)),
            pl.BlockSpec((f, 1), lambda i: (0, 0)),
            pl.BlockSpec((1, 1), lambda i: (0, 0)),
        ],
        out_specs=pl.BlockSpec((_TC_BLK,), lambda i: (i,)),
        out_shape=jax.ShapeDtypeStruct((n_out,), jnp.float32),
    )(table_t, wcol, bias11)


@functools.cache
def _sc_pick(B, nu, nm):
    b_per_w = B // NW
    n_chunks = b_per_w // _IDX_CHUNK
    mesh = plsc.VectorSubcoreMesh(core_axis_name="c", subcore_axis_name="s")

    @functools.partial(
        pl.kernel,
        mesh=mesh,
        compiler_params=pltpu.CompilerParams(
            needs_layout_passes=False, use_tc_tiling_on_sc=False),
        out_type=jax.ShapeDtypeStruct((B,), jnp.float32),
        scratch_types=[
            pltpu.VMEM((b_per_w,), jnp.int32),
            pltpu.VMEM((b_per_w,), jnp.int32),
            pltpu.VMEM((b_per_w,), jnp.int32),
            pltpu.VMEM((b_per_w,), jnp.int32),
            pltpu.VMEM((b_per_w, L), jnp.float32),
            pltpu.VMEM((b_per_w, L), jnp.float32),
            pltpu.VMEM((b_per_w,), jnp.float32),
            pltpu.SemaphoreType.DMA,
        ],
    )
    def k(uidx_hbm, midx_hbm, pu_hbm, pm_hbm, out_hbm,
          uid_v, mid_v, uhi_v, mhi_v, ubuf, mbuf, out_v, sem):
        wid = lax.axis_index("s") * NC + lax.axis_index("c")
        base = wid * b_per_w
        pltpu.sync_copy(uidx_hbm.at[pl.ds(base, b_per_w)], uid_v)
        pltpu.sync_copy(midx_hbm.at[pl.ds(base, b_per_w)], mid_v)
        for c in range(b_per_w // L):
            sl = pl.ds(c * L, L)
            uhi_v[sl] = lax.shift_right_logical(uid_v[sl], 4)
            mhi_v[sl] = lax.shift_right_logical(mid_v[sl], 4)
        copies = []
        for c in range(n_chunks):
            sl = pl.ds(c * _IDX_CHUNK, _IDX_CHUNK)
            copies.append(
                pltpu.async_copy(pu_hbm.at[uhi_v.at[sl]], ubuf.at[sl], sem))
            copies.append(
                pltpu.async_copy(pm_hbm.at[mhi_v.at[sl]], mbuf.at[sl], sem))
        for cp in copies:
            cp.wait()
        lanes = lax.iota(jnp.int32, L)
        for g in range(b_per_w // L):
            sl = pl.ds(g * L, L)
            rowv = lanes + (g * L)
            ulo = lax.bitwise_and(uid_v[sl], L - 1)
            mlo = lax.bitwise_and(mid_v[sl], L - 1)
            pu = plsc.load_gather(ubuf, [rowv, ulo])
            pm = plsc.load_gather(mbuf, [rowv, mlo])
            out_v[sl] = pu + pm
        pltpu.sync_copy(out_v, out_hbm.at[pl.ds(base, b_per_w)])

    return k


def kernel(user, movie, user_table, movie_table, W, b):
    B = user.shape[0]
    tu = user_table.T
    tm = movie_table.T
    wu = W[0, :N_FACTORS].reshape(N_FACTORS, 1)
    wm = W[0, N_FACTORS:].reshape(N_FACTORS, 1)
    pu, pm = _project_both(tu, tm, wu, wm, b.reshape(1, 1))
    p2u = pu.reshape(-1, L)
    p2m = pm.reshape(-1, L)
    out = _sc_pick(B, p2u.shape[0], p2m.shape[0])(
        user.astype(jnp.int32), movie.astype(jnp.int32), p2u, p2m)
    return out.reshape(B, 1)


# speedup vs baseline: 1.0427x; 1.0083x over previous
import functools

import jax
import jax.numpy as jnp
from jax import lax
from jax.experimental import pallas as pl
from jax.experimental.pallas import tpu as pltpu
from jax.experimental.pallas import tpu_sc as plsc

N_FACTORS = 64
L = 16

_info = plsc.get_sparse_core_info()
NC, NS = _info.num_cores, _info.num_subcores
NW = NC * NS

_IDX_CHUNK = 128
_TC_BLK = 32768


def _project_both(tu, tm, wu, wm, bias11):
    f, nu = tu.shape
    _, nm = tm.shape
    gu = (nu + _TC_BLK - 1) // _TC_BLK
    gm = (nm + _TC_BLK - 1) // _TC_BLK

    def body(tu_ref, tm_ref, wu_ref, wm_ref, b_ref, pu_ref, pm_ref):
        i = pl.program_id(0)

        @pl.when(i < gu)
        def _():
            pu_ref[...] = jnp.sum(tu_ref[...] * wu_ref[...], axis=0)

        @pl.when(i >= gu)
        def _():
            pm_ref[...] = (jnp.sum(tm_ref[...] * wm_ref[...], axis=0)
                           + b_ref[0, 0])

    return pl.pallas_call(
        body,
        grid=(gu + gm,),
        compiler_params=pltpu.CompilerParams(
            vmem_limit_bytes=60 * 1024 * 1024),
        in_specs=[
            pl.BlockSpec((f, _TC_BLK),
                         lambda i: (0, jnp.minimum(i, gu - 1))),
            pl.BlockSpec((f, _TC_BLK),
                         lambda i: (0, jnp.clip(i - gu, 0, gm - 1))),
            pl.BlockSpec((f, 1), lambda i: (0, 0)),
            pl.BlockSpec((f, 1), lambda i: (0, 0)),
            pl.BlockSpec((1, 1), lambda i: (0, 0)),
        ],
        out_specs=[
            pl.BlockSpec((_TC_BLK,), lambda i: (jnp.minimum(i, gu - 1),)),
            pl.BlockSpec((_TC_BLK,), lambda i: (jnp.clip(i - gu, 0, gm - 1),)),
        ],
        out_shape=[
            jax.ShapeDtypeStruct((nu,), jnp.float32),
            jax.ShapeDtypeStruct((nm,), jnp.float32),
        ],
    )(tu, tm, wu, wm, bias11)


@functools.cache
def _sc_pick(B, nu, nm):
    b_per_w = B // NW
    n_chunks = b_per_w // _IDX_CHUNK
    mesh = plsc.VectorSubcoreMesh(core_axis_name="c", subcore_axis_name="s")

    @functools.partial(
        pl.kernel,
        mesh=mesh,
        compiler_params=pltpu.CompilerParams(
            needs_layout_passes=False, use_tc_tiling_on_sc=False),
        out_type=jax.ShapeDtypeStruct((B,), jnp.float32),
        scratch_types=[
            pltpu.VMEM((b_per_w,), jnp.int32),
            pltpu.VMEM((b_per_w,), jnp.int32),
            pltpu.VMEM((b_per_w,), jnp.int32),
            pltpu.VMEM((b_per_w,), jnp.int32),
            pltpu.VMEM((b_per_w, L), jnp.float32),
            pltpu.VMEM((b_per_w, L), jnp.float32),
            pltpu.VMEM((b_per_w,), jnp.float32),
            pltpu.SemaphoreType.DMA,
        ],
    )
    def k(uidx_hbm, midx_hbm, pu_hbm, pm_hbm, out_hbm,
          uid_v, mid_v, uhi_v, mhi_v, ubuf, mbuf, out_v, sem):
        wid = lax.axis_index("s") * NC + lax.axis_index("c")
        base = wid * b_per_w
        pltpu.sync_copy(uidx_hbm.at[pl.ds(base, b_per_w)], uid_v)
        pltpu.sync_copy(midx_hbm.at[pl.ds(base, b_per_w)], mid_v)
        for c in range(b_per_w // L):
            sl = pl.ds(c * L, L)
            uhi_v[sl] = lax.shift_right_logical(uid_v[sl], 4)
            mhi_v[sl] = lax.shift_right_logical(mid_v[sl], 4)
        copies = []
        for c in range(n_chunks):
            sl = pl.ds(c * _IDX_CHUNK, _IDX_CHUNK)
            copies.append(
                pltpu.async_copy(pu_hbm.at[uhi_v.at[sl]], ubuf.at[sl], sem))
            copies.append(
                pltpu.async_copy(pm_hbm.at[mhi_v.at[sl]], mbuf.at[sl], sem))
        for cp in copies:
            cp.wait()
        lanes = lax.iota(jnp.int32, L)
        for g in range(b_per_w // L):
            sl = pl.ds(g * L, L)
            rowv = lanes + (g * L)
            ulo = lax.bitwise_and(uid_v[sl], L - 1)
            mlo = lax.bitwise_and(mid_v[sl], L - 1)
            pu = plsc.load_gather(ubuf, [rowv, ulo])
            pm = plsc.load_gather(mbuf, [rowv, mlo])
            out_v[sl] = pu + pm
        pltpu.sync_copy(out_v, out_hbm.at[pl.ds(base, b_per_w)])

    return k


def kernel(user, movie, user_table, movie_table, W, b):
    B = user.shape[0]
    tu = user_table.T
    tm = movie_table.T
    wu = W[0, :N_FACTORS].reshape(N_FACTORS, 1)
    wm = W[0, N_FACTORS:].reshape(N_FACTORS, 1)
    pu, pm = _project_both(tu, tm, wu, wm, b.reshape(1, 1))
    p2u = pu.reshape(-1, L)
    p2m = pm.reshape(-1, L)
    out = _sc_pick(B, p2u.shape[0], p2m.shape[0])(
        user.astype(jnp.int32), movie.astype(jnp.int32), p2u, p2m)
    return out.reshape(B, 1)
